# baseline (speedup 1.0000x reference)
import functools

import jax
import jax.numpy as jnp
from jax import lax
from jax.experimental import pallas as pl
from jax.experimental.pallas import tpu as pltpu
from jax.experimental.pallas import tpu_sc as plsc

D_MODEL = 64
NC, NS = 2, 16
NW = NC * NS
ROW = 128
K = 8
CHUNK = K * ROW


def _make_kernel(B: int):
  rows_total = B // ROW
  rows_per_w = rows_total // NW
  steps = rows_per_w // K

  mesh = plsc.VectorSubcoreMesh(core_axis_name="c", subcore_axis_name="s")

  @functools.partial(
      pl.kernel,
      mesh=mesh,
      compiler_params=pltpu.CompilerParams(use_tc_tiling_on_sc=False),
      out_type=jax.ShapeDtypeStruct((rows_total, ROW, D_MODEL), jnp.float32),
      scratch_types=[
          pltpu.VMEM((K, ROW), jnp.int32),
          pltpu.VMEM((K, ROW, D_MODEL), jnp.float32),
          pltpu.SemaphoreType.DMA,
      ],
  )
  def k(idx_hbm, table_hbm, out_hbm, idx_v, rows_v, gsem):
    wid = lax.axis_index("s") * NC + lax.axis_index("c")
    base_w = wid * rows_per_w

    def body(step, _):
      base = base_w + step * K
      pltpu.sync_copy(idx_hbm.at[pl.ds(base, K)], idx_v)
      copies = [
          pltpu.async_copy(table_hbm.at[idx_v.at[j]], rows_v.at[j], gsem)
          for j in range(K)
      ]
      for c in copies:
        c.wait()
      pltpu.sync_copy(rows_v, out_hbm.at[pl.ds(base, K)])
      return ()

    lax.fori_loop(0, steps, body, ())

  return k


def kernel(x, table):
  B = x.size
  idx = x.reshape(B // ROW, ROW).astype(jnp.int32)
  out = _make_kernel(B)(idx, table)
  return out.reshape(x.shape + (D_MODEL,))


# speedup vs baseline: 1.0099x; 1.0099x over previous
import functools

import jax
import jax.numpy as jnp
from jax import lax
from jax.experimental import pallas as pl
from jax.experimental.pallas import tpu as pltpu
from jax.experimental.pallas import tpu_sc as plsc

D_MODEL = 64
NC, NS = 2, 16
NW = NC * NS
ROW = 128
G = 5


def _make_kernel(B: int):
  rows_total = B // ROW
  rows_per_w = rows_total // NW
  groups = rows_per_w // G

  mesh = plsc.VectorSubcoreMesh(core_axis_name="c", subcore_axis_name="s")

  @functools.partial(
      pl.kernel,
      mesh=mesh,
      compiler_params=pltpu.CompilerParams(use_tc_tiling_on_sc=False),
      out_type=jax.ShapeDtypeStruct((rows_total, ROW, D_MODEL), jnp.float32),
      scratch_types=[
          pltpu.VMEM((2, G, ROW), jnp.int32),
          pltpu.VMEM((2, G, ROW, D_MODEL), jnp.float32),
          pltpu.SemaphoreType.DMA,
          pltpu.SemaphoreType.DMA,
      ],
  )
  def k(idx_hbm, table_hbm, out_hbm, idx_v, rows_v, gsem0, gsem1):
    wid = lax.axis_index("s") * NC + lax.axis_index("c")
    base_w = wid * rows_per_w
    sems = (gsem0, gsem1)

    def load_fire(b, g):
      pltpu.sync_copy(idx_hbm.at[pl.ds(base_w + g * G, G)], idx_v.at[b])
      for j in range(G):
        pltpu.async_copy(
            table_hbm.at[idx_v.at[b].at[j]], rows_v.at[b].at[j], sems[b])

    def drain(b):
      for j in range(G):
        pltpu.make_async_copy(
            table_hbm.at[idx_v.at[b].at[j]], rows_v.at[b].at[j],
            sems[b]).wait()

    def store(b, g):
      pltpu.sync_copy(rows_v.at[b], out_hbm.at[pl.ds(base_w + g * G, G)])

    load_fire(0, 0)

    def outer(o, _):
      g0 = 2 * o
      load_fire(1, g0 + 1)
      drain(0)
      store(0, g0)

      @pl.when(g0 + 2 < groups)
      def _():
        load_fire(0, g0 + 2)

      drain(1)
      store(1, g0 + 1)
      return ()

    lax.fori_loop(0, groups // 2, outer, ())

  return k


def kernel(x, table):
  B = x.size
  idx = x.reshape(B // ROW, ROW).astype(jnp.int32)
  out = _make_kernel(B)(idx, table)
  return out.reshape(x.shape + (D_MODEL,))


# speedup vs baseline: 1.3456x; 1.3324x over previous
import functools

import jax
import jax.numpy as jnp
from jax import lax
from jax.experimental import pallas as pl
from jax.experimental.pallas import tpu as pltpu
from jax.experimental.pallas import tpu_sc as plsc

D_MODEL = 64
NC, NS = 2, 16
NW = NC * NS
ROW = 128
G = 5


def _make_kernel(B: int):
  rows_total = B // ROW
  rows_per_w = rows_total // NW
  groups = rows_per_w // G

  mesh = plsc.VectorSubcoreMesh(core_axis_name="c", subcore_axis_name="s")

  @functools.partial(
      pl.kernel,
      mesh=mesh,
      compiler_params=pltpu.CompilerParams(use_tc_tiling_on_sc=False),
      out_type=jax.ShapeDtypeStruct((rows_total, ROW, 2 * D_MODEL), jnp.float32),
      scratch_types=[
          pltpu.VMEM((2, G, ROW), jnp.int32),
          pltpu.VMEM((2, G, ROW, D_MODEL), jnp.float32),
          pltpu.SemaphoreType.DMA,
          pltpu.SemaphoreType.DMA,
      ],
  )
  def k(idx_hbm, table_hbm, out_hbm, idx_v, rows_v, gsem0, gsem1):
    wid = lax.axis_index("s") * NC + lax.axis_index("c")
    base_w = wid * rows_per_w
    sems = (gsem0, gsem1)

    def load_fire(b, g):
      pltpu.sync_copy(idx_hbm.at[pl.ds(base_w + g * G, G)], idx_v.at[b])
      for j in range(G):
        pltpu.async_copy(
            table_hbm.at[idx_v.at[b].at[j]], rows_v.at[b].at[j], sems[b])

    def drain(b):
      for j in range(G):
        pltpu.make_async_copy(
            table_hbm.at[idx_v.at[b].at[j]], rows_v.at[b].at[j],
            sems[b]).wait()

    def store(b, g):
      pltpu.sync_copy(
          rows_v.at[b],
          out_hbm.at[pl.ds(base_w + g * G, G), :, pl.ds(0, D_MODEL)])

    load_fire(0, 0)

    def outer(o, _):
      g0 = 2 * o
      load_fire(1, g0 + 1)
      drain(0)
      store(0, g0)

      @pl.when(g0 + 2 < groups)
      def _():
        load_fire(0, g0 + 2)

      drain(1)
      store(1, g0 + 1)
      return ()

    lax.fori_loop(0, groups // 2, outer, ())

  return k


def kernel(x, table):
  B = x.size
  idx = x.reshape(B // ROW, ROW).astype(jnp.int32)
  out = _make_kernel(B)(idx, table)
  return out[:, :, :D_MODEL].reshape(x.shape + (D_MODEL,))
